# baseline (speedup 1.0000x reference)
import functools

import jax
import jax.numpy as jnp
from jax import lax
from jax.experimental import pallas as pl
from jax.experimental.pallas import tpu as pltpu
from jax.experimental.pallas import tpu_sc as plsc

VOCAB = 1000000
B = 16384
L = 200
D = 32
HID = 512
NCLS = 1000

NC = 2
NS = 16
NW = NC * NS
BAGS_PER_W = B // NW
G = 4
CHUNK = 100
CPR = G * L // CHUNK
ROUNDS = BAGS_PER_W // G
TEXT_ROWS_PER_ROUND = G * L // CHUNK


PACK_ROWS = 250000
PACK_VB = 12800
PACK_GRID = 79


def _tc_pack_table(table_t):
    R = PACK_VB // 4

    def body(x_ref, o_ref, xt_ref):
        r = lax.broadcasted_iota(jnp.int32, (D, D), 0)
        c = lax.broadcasted_iota(jnp.int32, (D, D), 1)
        eye = (r == c).astype(jnp.float32)
        xt_ref[...] = lax.dot_general(
            x_ref[...], eye, (((0,), (0,)), ((), ())),
            preferred_element_type=jnp.float32)
        for q in range(4):
            o_ref[:, q * D:(q + 1) * D] = xt_ref[pl.ds(q, R, 4), :]

    return pl.pallas_call(
        body,
        grid=(PACK_GRID,),
        in_specs=[pl.BlockSpec((D, PACK_VB), lambda i: (0, i))],
        out_specs=pl.BlockSpec((R, 4 * D), lambda i: (i, 0)),
        out_shape=jax.ShapeDtypeStruct((PACK_ROWS, 4 * D), jnp.float32),
        scratch_shapes=[pltpu.VMEM((PACK_VB, D), jnp.float32)],
    )(table_t)


def _sc_embedding_bag(text, emb_table):
    mesh = plsc.VectorSubcoreMesh(core_axis_name="c", subcore_axis_name="s")

    SPLITS = ((0, 104), (104, 96))

    @functools.partial(
        pl.kernel,
        out_type=jax.ShapeDtypeStruct((B, D), jnp.float32),
        mesh=mesh,
        scratch_types=[
            pltpu.VMEM((2, G, L), jnp.int32),
            pltpu.VMEM((2, G, L, D), jnp.float32),
            pltpu.VMEM((BAGS_PER_W, D), jnp.float32),
            pltpu.SemaphoreType.DMA,
            pltpu.SemaphoreType.DMA,
            pltpu.SemaphoreType.DMA,
            pltpu.SemaphoreType.DMA,
        ],
        compiler_params=pltpu.CompilerParams(use_tc_tiling_on_sc=False),
    )
    def body(text_hbm, table_hbm, out_hbm, idx_v, rows_v, emb_v,
             rs0, rs1, is0, is1):
        cid = lax.axis_index("c")
        sid = lax.axis_index("s")
        wid = sid * NC + cid
        rsem = [rs0, rs1]
        isem = [is0, is1]

        out_row0 = wid * BAGS_PER_W

        def idx_src(r):
            return text_hbm.at[pl.ds(out_row0 + r * G, G)]

        def fire_gathers(bank):
            for b in range(G):
                for off, sz in SPLITS:
                    pltpu.async_copy(
                        table_hbm.at[idx_v.at[bank, b, pl.ds(off, sz)]],
                        rows_v.at[bank, b, pl.ds(off, sz)], rsem[bank])

        def drain_gathers(bank):
            for b in range(G):
                for off, sz in SPLITS:
                    pltpu.make_async_copy(
                        table_hbm.at[idx_v.at[bank, b, pl.ds(off, sz)]],
                        rows_v.at[bank, b, pl.ds(off, sz)],
                        rsem[bank]).wait()

        def accumulate(bank, r):
            zero = jnp.zeros((16,), jnp.float32)
            init = (zero,) * (2 * G)

            def acc_row(rr, accs):
                accs = list(accs)
                for b in range(G):
                    for h in range(2):
                        v = rows_v[bank, b, rr, 16 * h:16 * h + 16]
                        accs[2 * b + h] = accs[2 * b + h] + v
                return tuple(accs)

            accs = lax.fori_loop(0, L, acc_row, init, unroll=2)
            for b in range(G):
                emb_v[r * G + b, 0:16] = accs[2 * b] * (1.0 / L)
                emb_v[r * G + b, 16:32] = accs[2 * b + 1] * (1.0 / L)

        def phase(r, a, b):
            @pl.when(r + 1 < ROUNDS)
            def _():
                pltpu.make_async_copy(idx_src(r + 1), idx_v.at[b],
                                      isem[b]).wait()
                fire_gathers(b)
            drain_gathers(a)

            @pl.when(r + 2 < ROUNDS)
            def _():
                pltpu.async_copy(idx_src(r + 2), idx_v.at[a], isem[a])

            accumulate(a, r)

        pltpu.async_copy(idx_src(0), idx_v.at[0], is0).wait()
        fire_gathers(0)
        pltpu.async_copy(idx_src(1), idx_v.at[1], is1)

        def gbody(g, carry):
            phase(2 * g, 0, 1)
            phase(2 * g + 1, 1, 0)
            return carry

        lax.fori_loop(0, ROUNDS // 2, gbody, 0)
        pltpu.sync_copy(emb_v, out_hbm.at[pl.ds(out_row0, BAGS_PER_W)])

    return body(text, emb_table)


def _tc_mlp_t(x, W1, b1, w2t, b2c):
    BT = 2048
    grid = (B // BT,)

    def body(x_ref, w1_ref, b1_ref, w2_ref, b2_ref, o_ref):
        h = jnp.dot(x_ref[...], w1_ref[...],
                    preferred_element_type=jnp.float32) + b1_ref[...]
        h = jnp.maximum(h, 0.0)
        z = lax.dot_general(w2_ref[...], h, (((1,), (1,)), ((), ())),
                            preferred_element_type=jnp.float32)
        z = z + b2_ref[...]
        o_ref[...] = 1.0 / (1.0 + jnp.exp(-z))

    return pl.pallas_call(
        body,
        grid=grid,
        in_specs=[
            pl.BlockSpec((BT, D), lambda i: (i, 0)),
            pl.BlockSpec((D, HID), lambda i: (0, 0)),
            pl.BlockSpec((1, HID), lambda i: (0, 0)),
            pl.BlockSpec((NCLS, HID), lambda i: (0, 0)),
            pl.BlockSpec((NCLS, 1), lambda i: (0, 0)),
        ],
        out_specs=pl.BlockSpec((NCLS, BT), lambda i: (0, i)),
        out_shape=jax.ShapeDtypeStruct((NCLS, B), jnp.float32),
    )(x, W1, b1, w2t, b2c)


def kernel(text, emb_table, W1, b1, W2, b2):
    table_lin = _tc_pack_table(emb_table.T).reshape(VOCAB, D)
    emb = _sc_embedding_bag(text, table_lin)
    out_t = _tc_mlp_t(emb, W1, b1.reshape(1, HID), W2.T,
                      b2.reshape(NCLS, 1))
    return out_t.T


# speedup vs baseline: 1.0330x; 1.0330x over previous
import functools

import jax
import jax.numpy as jnp
from jax import lax
from jax.experimental import pallas as pl
from jax.experimental.pallas import tpu as pltpu
from jax.experimental.pallas import tpu_sc as plsc

VOCAB = 1000000
B = 16384
L = 200
D = 32
HID = 512
NCLS = 1000

NC = 2
NS = 16
NW = NC * NS
BAGS_PER_W = B // NW
G = 4
CHUNK = 100
CPR = G * L // CHUNK
ROUNDS = BAGS_PER_W // G
TEXT_ROWS_PER_ROUND = G * L // CHUNK


PACK_ROWS = 250000
PACK_VB = 25600
PACK_GRID = 40


def _tc_pack_table(table_t):
    R = PACK_VB // 4

    def body(x_ref, o_ref, xt_ref):
        xt_ref[...] = x_ref[...].T
        for q in range(4):
            o_ref[:, q * D:(q + 1) * D] = xt_ref[pl.ds(q, R, 4), :]

    return pl.pallas_call(
        body,
        grid=(PACK_GRID,),
        in_specs=[pl.BlockSpec((D, PACK_VB), lambda i: (0, i))],
        out_specs=pl.BlockSpec((R, 4 * D), lambda i: (i, 0)),
        out_shape=jax.ShapeDtypeStruct((PACK_ROWS, 4 * D), jnp.float32),
        scratch_shapes=[pltpu.VMEM((PACK_VB, D), jnp.float32)],
    )(table_t)


def _sc_embedding_bag(text, emb_table):
    mesh = plsc.VectorSubcoreMesh(core_axis_name="c", subcore_axis_name="s")

    SPLITS = ((0, 104), (104, 96))

    @functools.partial(
        pl.kernel,
        out_type=jax.ShapeDtypeStruct((B, D), jnp.float32),
        mesh=mesh,
        scratch_types=[
            pltpu.VMEM((2, G, L), jnp.int32),
            pltpu.VMEM((2, G, L, D), jnp.float32),
            pltpu.VMEM((BAGS_PER_W, D), jnp.float32),
            pltpu.SemaphoreType.DMA,
            pltpu.SemaphoreType.DMA,
            pltpu.SemaphoreType.DMA,
            pltpu.SemaphoreType.DMA,
        ],
        compiler_params=pltpu.CompilerParams(use_tc_tiling_on_sc=False),
    )
    def body(text_hbm, table_hbm, out_hbm, idx_v, rows_v, emb_v,
             rs0, rs1, is0, is1):
        cid = lax.axis_index("c")
        sid = lax.axis_index("s")
        wid = sid * NC + cid
        rsem = [rs0, rs1]
        isem = [is0, is1]

        out_row0 = wid * BAGS_PER_W

        def idx_src(r):
            return text_hbm.at[pl.ds(out_row0 + r * G, G)]

        def fire_gathers(bank):
            for b in range(G):
                for off, sz in SPLITS:
                    pltpu.async_copy(
                        table_hbm.at[idx_v.at[bank, b, pl.ds(off, sz)]],
                        rows_v.at[bank, b, pl.ds(off, sz)], rsem[bank])

        def drain_gathers(bank):
            for b in range(G):
                for off, sz in SPLITS:
                    pltpu.make_async_copy(
                        table_hbm.at[idx_v.at[bank, b, pl.ds(off, sz)]],
                        rows_v.at[bank, b, pl.ds(off, sz)],
                        rsem[bank]).wait()

        def accumulate(bank, r):
            zero = jnp.zeros((16,), jnp.float32)
            init = (zero,) * (2 * G)

            def acc_row(rr, accs):
                accs = list(accs)
                for b in range(G):
                    for h in range(2):
                        v = rows_v[bank, b, rr, 16 * h:16 * h + 16]
                        accs[2 * b + h] = accs[2 * b + h] + v
                return tuple(accs)

            accs = lax.fori_loop(0, L, acc_row, init, unroll=2)
            for b in range(G):
                emb_v[r * G + b, 0:16] = accs[2 * b] * (1.0 / L)
                emb_v[r * G + b, 16:32] = accs[2 * b + 1] * (1.0 / L)

        def phase(r, a, b):
            @pl.when(r + 1 < ROUNDS)
            def _():
                pltpu.make_async_copy(idx_src(r + 1), idx_v.at[b],
                                      isem[b]).wait()
                fire_gathers(b)
            drain_gathers(a)

            @pl.when(r + 2 < ROUNDS)
            def _():
                pltpu.async_copy(idx_src(r + 2), idx_v.at[a], isem[a])

            accumulate(a, r)

        pltpu.async_copy(idx_src(0), idx_v.at[0], is0).wait()
        fire_gathers(0)
        pltpu.async_copy(idx_src(1), idx_v.at[1], is1)

        def gbody(g, carry):
            phase(2 * g, 0, 1)
            phase(2 * g + 1, 1, 0)
            return carry

        lax.fori_loop(0, ROUNDS // 2, gbody, 0)
        pltpu.sync_copy(emb_v, out_hbm.at[pl.ds(out_row0, BAGS_PER_W)])

    return body(text, emb_table)


def _tc_mlp_t(x, W1, b1, w2t, b2c):
    BT = 2048
    grid = (B // BT,)

    def body(x_ref, w1_ref, b1_ref, w2_ref, b2_ref, o_ref):
        h = jnp.dot(x_ref[...], w1_ref[...],
                    preferred_element_type=jnp.float32) + b1_ref[...]
        h = jnp.maximum(h, 0.0)
        z = lax.dot_general(w2_ref[...], h, (((1,), (1,)), ((), ())),
                            preferred_element_type=jnp.float32)
        z = z + b2_ref[...]
        o_ref[...] = 1.0 / (1.0 + jnp.exp(-z))

    return pl.pallas_call(
        body,
        grid=grid,
        in_specs=[
            pl.BlockSpec((BT, D), lambda i: (i, 0)),
            pl.BlockSpec((D, HID), lambda i: (0, 0)),
            pl.BlockSpec((1, HID), lambda i: (0, 0)),
            pl.BlockSpec((NCLS, HID), lambda i: (0, 0)),
            pl.BlockSpec((NCLS, 1), lambda i: (0, 0)),
        ],
        out_specs=pl.BlockSpec((NCLS, BT), lambda i: (0, i)),
        out_shape=jax.ShapeDtypeStruct((NCLS, B), jnp.float32),
    )(x, W1, b1, w2t, b2c)


def kernel(text, emb_table, W1, b1, W2, b2):
    table_lin = _tc_pack_table(emb_table.T).reshape(VOCAB, D)
    emb = _sc_embedding_bag(text, table_lin)
    out_t = _tc_mlp_t(emb, W1, b1.reshape(1, HID), W2.T,
                      b2.reshape(NCLS, 1))
    return out_t.T
